# Initial kernel scaffold; baseline (speedup 1.0000x reference)
import jax
import jax.numpy as jnp
from jax.experimental import pallas as pl


def kernel(x, W):
    raise NotImplementedError("write your pallas kernel here")


# speedup vs baseline: 83.1825x; 83.1825x over previous
import functools

import jax
import jax.numpy as jnp
import numpy as np
from jax.experimental import pallas as pl
from jax.experimental.pallas import tpu as pltpu

_INF = float(np.float32(3.0e38))


def _roll_l(x, s):
  n = x.shape[-1]
  s = s % n
  if s == 0:
    return x
  return jnp.concatenate([x[..., n - s:], x[..., :n - s]], axis=-1)


def _sparsify_body(w_ref, wsoft_ref, stats_ref, colabs, keep):
  p = pl.program_id(0)
  t = pl.program_id(1)
  n = w_ref.shape[-1]

  @pl.when((p == 0) & (t == 0))
  def _init():
    colabs[...] = jnp.zeros_like(colabs)
    stats_ref[0] = 0.0
    stats_ref[1] = 0.0

  @pl.when(p == 0)
  def _acc_colabs():
    colabs[...] += jnp.sum(jnp.abs(w_ref[...]), axis=0, keepdims=True)

  @pl.when((p == 1) & (t == 0))
  def _make_keep():
    c = colabs[...]
    ob = jax.lax.broadcasted_iota(jnp.int32, (1, n), 1) % 8
    rank = jnp.zeros((1, n), jnp.float32)
    for k in range(1, 8):
      p_nw = _roll_l(c, -k)
      p_w = _roll_l(c, 8 - k)
      nw = ob < (8 - k)
      beat_nw = jnp.where(p_nw > c, 1.0, 0.0)
      beat_w = jnp.where(p_w >= c, 1.0, 0.0)
      rank += jnp.where(nw, beat_nw, beat_w)
    keep[...] = jnp.where(rank < 3.5, 1.0, 0.0)

  @pl.when(p == 1)
  def _soft():
    w = w_ref[...]
    r = w.shape[0]
    aw = jnp.abs(w)
    kb = jnp.broadcast_to(keep[...], (r, n)) > 0.5
    ob = jax.lax.broadcasted_iota(jnp.int32, (r, n), 1) % 8
    m = jnp.where(kb, aw, _INF)
    m1 = m
    m2 = jnp.full_like(m, _INF)
    for d in (1, 2, 4):
      valid = ob < (8 - d)
      b1 = jnp.where(valid, _roll_l(m1, -d), _INF)
      b2 = jnp.where(valid, _roll_l(m2, -d), _INF)
      hi = jnp.maximum(m1, b1)
      m1 = jnp.minimum(m1, b1)
      m2 = jnp.minimum(hi, jnp.minimum(m2, b2))
    thr = m2
    for d in (1, 2, 4):
      use_left = (ob & d) != 0
      thr = jnp.where(use_left, _roll_l(thr, d), thr)
    v = jnp.maximum(aw - thr, 0.0)
    sv = jnp.where(w >= 0, v, -v)
    ws = jnp.where(kb, sv, 0.0)
    stats_ref[0] += jnp.sum(w * ws)
    stats_ref[1] += jnp.sum(ws * ws)
    wsoft_ref[...] = ws.astype(jnp.bfloat16)


def _matmul_body(beta_ref, x_ref, w_ref, y_ref):
  xb = x_ref[...].astype(jnp.bfloat16)
  acc = jax.lax.dot_general(
      xb, w_ref[...],
      dimension_numbers=(((1,), (1,)), ((), ())),
      preferred_element_type=jnp.float32)
  y_ref[...] = acc * beta_ref[0]


def _pick(n, t):
  return t if n % t == 0 else n


def _impl(x, W, interpret):
  b, s, d = x.shape
  out_f, in_f = W.shape
  xm = x.reshape(b * s, d)
  mm = b * s

  to = _pick(out_f, 128)
  nt = out_f // to

  wsoft, stats = pl.pallas_call(
      _sparsify_body,
      grid=(2, nt),
      in_specs=[pl.BlockSpec((to, in_f), lambda p, t: (t, 0))],
      out_specs=[
          pl.BlockSpec((to, in_f), lambda p, t: (t, 0)),
          pl.BlockSpec(memory_space=pltpu.SMEM),
      ],
      out_shape=[
          jax.ShapeDtypeStruct((out_f, in_f), jnp.bfloat16),
          jax.ShapeDtypeStruct((2,), jnp.float32),
      ],
      scratch_shapes=[
          pltpu.VMEM((1, in_f), jnp.float32),
          pltpu.VMEM((1, in_f), jnp.float32),
      ],
      compiler_params=pltpu.CompilerParams(
          dimension_semantics=("arbitrary", "arbitrary"),
          vmem_limit_bytes=100 * 1024 * 1024,
      ),
      interpret=interpret,
  )(W)

  num = stats[0]
  den = stats[1]
  beta = jnp.where(den > 0, num / den, jnp.float32(1.0))
  beta = jax.lax.stop_gradient(beta).reshape(1)

  bm = _pick(mm, 512)
  bn = _pick(out_f, 512)
  y = pl.pallas_call(
      _matmul_body,
      grid=(mm // bm, out_f // bn),
      in_specs=[
          pl.BlockSpec(memory_space=pltpu.SMEM),
          pl.BlockSpec((bm, in_f), lambda i, j: (i, 0)),
          pl.BlockSpec((bn, in_f), lambda i, j: (j, 0)),
      ],
      out_specs=pl.BlockSpec((bm, bn), lambda i, j: (i, j)),
      out_shape=jax.ShapeDtypeStruct((mm, out_f), jnp.float32),
      compiler_params=pltpu.CompilerParams(
          dimension_semantics=("parallel", "arbitrary"),
          vmem_limit_bytes=100 * 1024 * 1024,
      ),
      interpret=interpret,
  )(beta, xm, wsoft)

  return y.reshape(b, s, out_f)


def kernel(x, W):
  return _impl(x, W, False)


# speedup vs baseline: 86.2739x; 1.0372x over previous
import functools

import jax
import jax.numpy as jnp
import numpy as np
from jax.experimental import pallas as pl
from jax.experimental.pallas import tpu as pltpu

_INF = float(np.float32(3.0e38))


def _roll_l(x, s):
  n = x.shape[-1]
  s = s % n
  if s == 0:
    return x
  return jnp.concatenate([x[..., n - s:], x[..., :n - s]], axis=-1)


def _sparsify_body(w_ref, wsoft_ref, stats_ref, colabs, keep):
  p = pl.program_id(0)
  t = pl.program_id(1)
  n = w_ref.shape[-1]

  @pl.when((p == 0) & (t == 0))
  def _init():
    colabs[...] = jnp.zeros_like(colabs)
    stats_ref[0] = 0.0
    stats_ref[1] = 0.0

  @pl.when(p == 0)
  def _acc_colabs():
    colabs[...] += jnp.sum(jnp.abs(w_ref[...]), axis=0, keepdims=True)

  @pl.when((p == 1) & (t == 0))
  def _make_keep():
    c = colabs[...]
    ob = jax.lax.broadcasted_iota(jnp.int32, (1, n), 1) % 8
    rank = jnp.zeros((1, n), jnp.float32)
    for k in range(1, 8):
      p_nw = _roll_l(c, -k)
      p_w = _roll_l(c, 8 - k)
      nw = ob < (8 - k)
      beat_nw = jnp.where(p_nw > c, 1.0, 0.0)
      beat_w = jnp.where(p_w >= c, 1.0, 0.0)
      rank += jnp.where(nw, beat_nw, beat_w)
    keep[...] = jnp.where(rank < 3.5, 1.0, 0.0)

  @pl.when(p == 1)
  def _soft():
    w = w_ref[...]
    r = w.shape[0]
    aw = jnp.abs(w)
    kb = jnp.broadcast_to(keep[...], (r, n)) > 0.5
    ob = jax.lax.broadcasted_iota(jnp.int32, (r, n), 1) % 8
    m = jnp.where(kb, aw, _INF)
    m1 = m
    m2 = jnp.full_like(m, _INF)
    for d in (1, 2, 4):
      valid = ob < (8 - d)
      b1 = jnp.where(valid, _roll_l(m1, -d), _INF)
      b2 = jnp.where(valid, _roll_l(m2, -d), _INF)
      hi = jnp.maximum(m1, b1)
      m1 = jnp.minimum(m1, b1)
      m2 = jnp.minimum(hi, jnp.minimum(m2, b2))
    thr = m2
    for d in (1, 2, 4):
      use_left = (ob & d) != 0
      thr = jnp.where(use_left, _roll_l(thr, d), thr)
    v = jnp.maximum(aw - thr, 0.0)
    sv = jnp.where(w >= 0, v, -v)
    ws = jnp.where(kb, sv, 0.0)
    stats_ref[0] += jnp.sum(w * ws)
    stats_ref[1] += jnp.sum(ws * ws)
    wsoft_ref[...] = ws.astype(jnp.bfloat16)


def _cast_body(x_ref, o_ref):
  o_ref[...] = x_ref[...].astype(jnp.bfloat16)


def _matmul_body(beta_ref, x_ref, w_ref, y_ref):
  acc = jax.lax.dot_general(
      x_ref[...], w_ref[...],
      dimension_numbers=(((1,), (1,)), ((), ())),
      preferred_element_type=jnp.float32)
  y_ref[...] = acc * beta_ref[0]


def _pick(n, t):
  return t if n % t == 0 else n


def _impl(x, W, interpret):
  b, s, d = x.shape
  out_f, in_f = W.shape
  xm = x.reshape(b * s, d)
  mm = b * s

  to = _pick(out_f, 128)
  nt = out_f // to

  wsoft, stats = pl.pallas_call(
      _sparsify_body,
      grid=(2, nt),
      in_specs=[pl.BlockSpec((to, in_f), lambda p, t: (t, 0))],
      out_specs=[
          pl.BlockSpec((to, in_f), lambda p, t: (t, 0)),
          pl.BlockSpec(memory_space=pltpu.SMEM),
      ],
      out_shape=[
          jax.ShapeDtypeStruct((out_f, in_f), jnp.bfloat16),
          jax.ShapeDtypeStruct((2,), jnp.float32),
      ],
      scratch_shapes=[
          pltpu.VMEM((1, in_f), jnp.float32),
          pltpu.VMEM((1, in_f), jnp.float32),
      ],
      compiler_params=pltpu.CompilerParams(
          dimension_semantics=("arbitrary", "arbitrary"),
          vmem_limit_bytes=100 * 1024 * 1024,
      ),
      interpret=interpret,
  )(W)

  num = stats[0]
  den = stats[1]
  beta = jnp.where(den > 0, num / den, jnp.float32(1.0))
  beta = jax.lax.stop_gradient(beta).reshape(1)

  bc = _pick(mm, 512)
  xb = pl.pallas_call(
      _cast_body,
      grid=(mm // bc,),
      in_specs=[pl.BlockSpec((bc, in_f), lambda i: (i, 0))],
      out_specs=pl.BlockSpec((bc, in_f), lambda i: (i, 0)),
      out_shape=jax.ShapeDtypeStruct((mm, in_f), jnp.bfloat16),
      compiler_params=pltpu.CompilerParams(
          dimension_semantics=("parallel",),
          vmem_limit_bytes=50 * 1024 * 1024,
      ),
      interpret=interpret,
  )(xm)

  bm = _pick(mm, 1024)
  bn = _pick(out_f, 1024)
  y = pl.pallas_call(
      _matmul_body,
      grid=(mm // bm, out_f // bn),
      in_specs=[
          pl.BlockSpec(memory_space=pltpu.SMEM),
          pl.BlockSpec((bm, in_f), lambda i, j: (i, 0)),
          pl.BlockSpec((bn, in_f), lambda i, j: (j, 0)),
      ],
      out_specs=pl.BlockSpec((bm, bn), lambda i, j: (i, j)),
      out_shape=jax.ShapeDtypeStruct((mm, out_f), jnp.float32),
      compiler_params=pltpu.CompilerParams(
          dimension_semantics=("parallel", "arbitrary"),
          vmem_limit_bytes=60 * 1024 * 1024,
      ),
      interpret=interpret,
  )(beta, xb, wsoft)

  return y.reshape(b, s, out_f)


def kernel(x, W):
  return _impl(x, W, False)
